# Initial kernel scaffold; baseline (speedup 1.0000x reference)
import jax
import jax.numpy as jnp
from jax.experimental import pallas as pl


def kernel(z, embedding):
    raise NotImplementedError("write your pallas kernel here")


# speedup vs baseline: 1.0883x; 1.0883x over previous
import functools

import jax
import jax.numpy as jnp
from jax import lax
from jax.experimental import pallas as pl
from jax.experimental.pallas import tpu as pltpu
from jax.experimental.pallas import tpu_sc as plsc


def _bf16_rtne(x):
    u = lax.bitcast_convert_type(x, jnp.int32)
    r = u + ((u >> 16) & 1) + 0x7FFF
    return lax.bitcast_convert_type(r & jnp.int32(-65536), jnp.float32)


def _argmin_body(eb, xbT_ref, emb_ref, zsq_ref, esq_ref, idx_ref,
                 bv_ref, bi_ref):
    e = pl.program_id(1)
    ne = pl.num_programs(1)

    xt = xbT_ref[...]
    w = emb_ref[...]
    d = lax.dot_general(w, xt, (((1,), (0,)), ((), ())),
                        preferred_element_type=jnp.float32)
    s = (zsq_ref[...] - d) + esq_ref[...]
    s = jnp.sqrt(jnp.maximum(s, 0.0))

    m = jnp.min(s, axis=0, keepdims=True)
    row = lax.broadcasted_iota(jnp.int32, s.shape, 0)
    li = jnp.min(jnp.where(s == m, row, eb), axis=0, keepdims=True)
    gi = li + e * eb

    @pl.when(e == 0)
    def _():
        bv_ref[...] = _bf16_rtne(m)
        bi_ref[...] = gi

    @pl.when(e > 0)
    def _():
        bv = bv_ref[...]
        better = m < bv
        sel = jnp.where(better, m, bv)
        bv_ref[...] = _bf16_rtne(sel)
        bi_ref[...] = jnp.where(better, gi, bi_ref[...])

    @pl.when(e == ne - 1)
    def _():
        idx_ref[...] = bi_ref[...]


def _tc_argmin(xbT, emb, zsq, esq, *, block_t=1024, block_e=4096,
               interpret=False):
    k, n = xbT.shape
    e_tot = emb.shape[0]
    nt = n // block_t
    ne = e_tot // block_e
    return pl.pallas_call(
        functools.partial(_argmin_body, block_e),
        grid=(nt, ne),
        in_specs=[
            pl.BlockSpec((k, block_t), lambda t, e: (0, t)),
            pl.BlockSpec((block_e, k), lambda t, e: (e, 0)),
            pl.BlockSpec((1, block_t), lambda t, e: (0, t)),
            pl.BlockSpec((block_e, 1), lambda t, e: (e, 0)),
        ],
        out_specs=pl.BlockSpec((1, block_t), lambda t, e: (0, t)),
        out_shape=jax.ShapeDtypeStruct((1, n), jnp.int32),
        scratch_shapes=[
            pltpu.VMEM((1, block_t), jnp.float32),
            pltpu.VMEM((1, block_t), jnp.int32),
        ],
        interpret=interpret,
    )(xbT, emb, zsq, esq)


def _sc_gather(idx, table):
    v, dm = table.shape
    bn = idx.shape[0]
    info = plsc.get_sparse_core_info()
    nc, ns = info.num_cores, info.num_subcores
    nw = nc * ns
    b_per_w = bn // nw
    mesh = plsc.VectorSubcoreMesh(core_axis_name="c", subcore_axis_name="s")

    @functools.partial(
        pl.kernel, mesh=mesh,
        out_type=jax.ShapeDtypeStruct((bn, dm), jnp.float32),
        scratch_types=[
            pltpu.VMEM((b_per_w,), jnp.int32),
            pltpu.VMEM((b_per_w, dm), jnp.float32),
            pltpu.SemaphoreType.DMA,
        ],
        compiler_params=pltpu.CompilerParams(use_tc_tiling_on_sc=False),
    )
    def gk(idx_hbm, table_hbm, out_hbm, idx_v, rows_v, sem):
        wid = lax.axis_index("s") * nc + lax.axis_index("c")
        base = wid * b_per_w
        pltpu.sync_copy(idx_hbm.at[pl.ds(base, b_per_w)], idx_v)
        pltpu.async_copy(table_hbm.at[idx_v], rows_v, sem).wait()
        pltpu.sync_copy(rows_v, out_hbm.at[pl.ds(base, b_per_w)])

    return gk(idx, table)


def kernel(z, embedding):
    b, c, d, h, w = z.shape
    flat2 = jnp.transpose(2.0 * z, (0, 2, 3, 4, 1)).reshape(-1, c)
    xbT = flat2.astype(jnp.bfloat16).T
    zsq = jnp.sum(z ** 2, axis=1).reshape(1, -1)
    esq = jnp.sum(embedding ** 2, axis=1)[:, None]

    idx2 = _tc_argmin(xbT, embedding, zsq, esq)
    idx = idx2.reshape(-1)

    q = _sc_gather(idx, embedding)

    quantized = jnp.transpose(q.reshape(b, d, h, w, c), (0, 4, 1, 2, 3))
    quantized_st = z + (quantized - z)
    loss = jnp.mean((quantized_st - z) ** 2)
    encoding_indices = idx.reshape(b, d, h, w)
    return quantized_st, loss, encoding_indices


# speedup vs baseline: 1.1131x; 1.0228x over previous
import functools

import jax
import jax.numpy as jnp
from jax import lax
from jax.experimental import pallas as pl
from jax.experimental.pallas import tpu as pltpu
from jax.experimental.pallas import tpu_sc as plsc


def _bf16_rtne(x):
    u = lax.bitcast_convert_type(x, jnp.int32)
    r = u + ((u >> 16) & 1) + 0x7FFF
    return lax.bitcast_convert_type(r & jnp.int32(-65536), jnp.float32)


def _argmin_body(eb, xbT_ref, emb_ref, zsq_ref, esq_ref, idx_ref,
                 bv_ref, bi_ref):
    e = pl.program_id(1)
    ne = pl.num_programs(1)

    xt = xbT_ref[...]
    w = emb_ref[...]
    d = lax.dot_general(w, xt, (((1,), (0,)), ((), ())),
                        preferred_element_type=jnp.float32)
    s = (zsq_ref[...] - d) + esq_ref[...]
    s = jnp.sqrt(jnp.maximum(s, 0.0))

    m = jnp.min(s, axis=0, keepdims=True)
    row = lax.broadcasted_iota(jnp.int32, s.shape, 0)
    li = jnp.min(jnp.where(s == m, row, eb), axis=0, keepdims=True)
    gi = li + e * eb

    @pl.when(e == 0)
    def _():
        bv_ref[...] = _bf16_rtne(m)
        bi_ref[...] = gi

    @pl.when(e > 0)
    def _():
        bv = bv_ref[...]
        better = m < bv
        sel = jnp.where(better, m, bv)
        bv_ref[...] = _bf16_rtne(sel)
        bi_ref[...] = jnp.where(better, gi, bi_ref[...])

    @pl.when(e == ne - 1)
    def _():
        idx_ref[...] = bi_ref[...]


def _tc_argmin(xbT, emb, zsq, esq, *, block_t=2048, block_e=4096,
               interpret=False):
    k, n = xbT.shape
    e_tot = emb.shape[0]
    nt = n // block_t
    ne = e_tot // block_e
    return pl.pallas_call(
        functools.partial(_argmin_body, block_e),
        grid=(nt, ne),
        in_specs=[
            pl.BlockSpec((k, block_t), lambda t, e: (0, t)),
            pl.BlockSpec((block_e, k), lambda t, e: (e, 0)),
            pl.BlockSpec((1, block_t), lambda t, e: (0, t)),
            pl.BlockSpec((block_e, 1), lambda t, e: (e, 0)),
        ],
        out_specs=pl.BlockSpec((1, block_t), lambda t, e: (0, t)),
        out_shape=jax.ShapeDtypeStruct((1, n), jnp.int32),
        scratch_shapes=[
            pltpu.VMEM((1, block_t), jnp.float32),
            pltpu.VMEM((1, block_t), jnp.int32),
        ],
        interpret=interpret,
    )(xbT, emb, zsq, esq)


def _sc_gather(idx, table):
    v, dm = table.shape
    bn = idx.shape[0]
    info = plsc.get_sparse_core_info()
    nc, ns = info.num_cores, info.num_subcores
    nw = nc * ns
    b_per_w = bn // nw
    mesh = plsc.VectorSubcoreMesh(core_axis_name="c", subcore_axis_name="s")

    @functools.partial(
        pl.kernel, mesh=mesh,
        out_type=jax.ShapeDtypeStruct((bn, dm), jnp.float32),
        scratch_types=[
            pltpu.VMEM((b_per_w,), jnp.int32),
            pltpu.VMEM((b_per_w, dm), jnp.float32),
            pltpu.SemaphoreType.DMA,
        ],
        compiler_params=pltpu.CompilerParams(use_tc_tiling_on_sc=False),
    )
    def gk(idx_hbm, table_hbm, out_hbm, idx_v, rows_v, sem):
        wid = lax.axis_index("s") * nc + lax.axis_index("c")
        base = wid * b_per_w
        pltpu.sync_copy(idx_hbm.at[pl.ds(base, b_per_w)], idx_v)
        pltpu.async_copy(table_hbm.at[idx_v], rows_v, sem).wait()
        pltpu.sync_copy(rows_v, out_hbm.at[pl.ds(base, b_per_w)])

    return gk(idx, table)


def kernel(z, embedding):
    b, c, d, h, w = z.shape
    flat2 = jnp.transpose(2.0 * z, (0, 2, 3, 4, 1)).reshape(-1, c)
    xbT = flat2.astype(jnp.bfloat16).T
    zsq = jnp.sum(z ** 2, axis=1).reshape(1, -1)
    esq = jnp.sum(embedding ** 2, axis=1)[:, None]

    idx2 = _tc_argmin(xbT, embedding, zsq, esq)
    idx = idx2.reshape(-1)

    q = _sc_gather(idx, embedding)

    quantized = jnp.transpose(q.reshape(b, d, h, w, c), (0, 4, 1, 2, 3))
    quantized_st = z + (quantized - z)
    loss = jnp.mean((quantized_st - z) ** 2)
    encoding_indices = idx.reshape(b, d, h, w)
    return quantized_st, loss, encoding_indices
